# baseline (speedup 1.0000x reference)
import functools
import jax
import jax.numpy as jnp
from jax import lax
from jax.experimental import pallas as pl
from jax.experimental.pallas import tpu as pltpu
from jax.experimental.pallas import tpu_sc as plsc

D = 64
ROWS = 16384
COLS = 20
B = ROWS * COLS
NC, NS, L = 2, 16, 16
NW = NC * NS
IPW = B // (NW * 128)
OPW = B // (NW * 2)
SCALE = 8.0

_mesh = plsc.VectorSubcoreMesh(core_axis_name="c", subcore_axis_name="s")


@functools.partial(
    pl.kernel,
    out_type=jax.ShapeDtypeStruct((B // 2, 2 * D), jnp.float32),
    mesh=_mesh,
    scratch_types=[
        pltpu.VMEM((IPW, 128), jnp.int32),
        pltpu.VMEM((128, 2 * D), jnp.float32),
        pltpu.VMEM((64, 2 * D), jnp.float32),
        pltpu.SemaphoreType.DMA,
    ],
    compiler_params=pltpu.CompilerParams(use_tc_tiling_on_sc=True),
)
def _emb(x_hbm, table_hbm, out_hbm, idx_v, rows_v, out_c, sem):
    wid = lax.axis_index("s") * NC + lax.axis_index("c")
    pltpu.sync_copy(x_hbm.at[pl.ds(wid * IPW, IPW)], idx_v)

    def vec_body(r, _):
        pltpu.async_copy(table_hbm.at[idx_v.at[r]], rows_v, sem).wait()

        def pack_row(q, _):
            for j in range(D // L):
                sl = pl.ds(j * L, L)
                sh = pl.ds(D + j * L, L)
                out_c[q, sl] = rows_v[2 * q, sl] * SCALE
                out_c[q, sh] = rows_v[2 * q + 1, sl] * SCALE
            return 0

        lax.fori_loop(0, 64, pack_row, 0)
        pltpu.sync_copy(out_c, out_hbm.at[pl.ds(wid * OPW + r * 64, 64)])
        return 0

    lax.fori_loop(0, IPW, vec_body, 0)


def kernel(x, table):
    table_p = jnp.pad(table, ((0, 0), (0, 128 - D)))
    x128 = x.astype(jnp.int32).reshape(B // 128, 128)
    out2 = _emb(x128, table_p)
    return out2.reshape(ROWS, COLS, D)
